# Initial kernel scaffold; baseline (speedup 1.0000x reference)
import jax
import jax.numpy as jnp
from jax.experimental import pallas as pl


def kernel(hidden_states, router_w, expert_gate_w, expert_up_w, expert_down_w, shared_gate_w, shared_up_w, shared_down_w, shared_expert_gate_w):
    raise NotImplementedError("write your pallas kernel here")


# speedup vs baseline: 1.2858x; 1.2858x over previous
import functools

import jax
import jax.numpy as jnp
from jax.experimental import pallas as pl
from jax.experimental.pallas import tpu as pltpu

_TOPK = 8


def _moe_body(xT_ref, xTb_ref, rw_ref, wg_ref, wu_ref, wd_ref,
              sg_ref, su_ref, sd_ref, sgw_ref,
              outT_ref, logitsT_ref, accT_ref, WT_ref):
    e = pl.program_id(0)
    n_e = pl.num_programs(0)
    E, T = logitsT_ref.shape

    @pl.when(e == 0)
    def _init():
        xT = xT_ref[...]
        xTb = xTb_ref[...]
        logitsT = jax.lax.dot(rw_ref[...], xT,
                              preferred_element_type=jnp.float32)
        logitsT_ref[...] = logitsT
        mx = jnp.max(logitsT, axis=0, keepdims=True)
        p = jnp.exp(logitsT - mx)
        p = p / jnp.sum(p, axis=0, keepdims=True)
        rowid = jax.lax.broadcasted_iota(jnp.int32, (E, T), 0)
        selp = jnp.zeros((E, T), jnp.float32)
        work = p
        for _ in range(_TOPK):
            cur = jnp.max(work, axis=0, keepdims=True)
            cand = jnp.where(work >= cur, rowid, E)
            first = jnp.min(cand, axis=0, keepdims=True)
            hit = rowid == first
            selp = jnp.where(hit, p, selp)
            work = jnp.where(hit, -jnp.inf, work)
        WT_ref[...] = selp / jnp.sum(selp, axis=0, keepdims=True)
        sgT = jax.lax.dot(sg_ref[...].astype(jnp.bfloat16), xTb,
                          preferred_element_type=jnp.float32)
        suT = jax.lax.dot(su_ref[...].astype(jnp.bfloat16), xTb,
                          preferred_element_type=jnp.float32)
        shT = (sgT * jax.nn.sigmoid(sgT)) * suT
        sdT = jax.lax.dot(sd_ref[...].astype(jnp.bfloat16),
                          shT.astype(jnp.bfloat16),
                          preferred_element_type=jnp.float32)
        gate = jax.lax.dot(sgw_ref[...], xT,
                           preferred_element_type=jnp.float32)
        accT_ref[...] = sdT * jax.nn.sigmoid(gate)

    xTb = xTb_ref[...]
    gT = jax.lax.dot(wg_ref[0].astype(jnp.bfloat16), xTb,
                     preferred_element_type=jnp.float32)
    uT = jax.lax.dot(wu_ref[0].astype(jnp.bfloat16), xTb,
                     preferred_element_type=jnp.float32)
    hT = (gT * jax.nn.sigmoid(gT)) * uT
    oT = jax.lax.dot(wd_ref[0].astype(jnp.bfloat16), hT.astype(jnp.bfloat16),
                     preferred_element_type=jnp.float32)
    we = WT_ref[pl.ds(e, 1), :]
    accT_ref[...] += oT * we

    @pl.when(e == n_e - 1)
    def _fin():
        outT_ref[...] = accT_ref[...]


@functools.partial(jax.jit, static_argnames=())
def kernel(hidden_states, router_w, expert_gate_w, expert_up_w,
           expert_down_w, shared_gate_w, shared_up_w, shared_down_w,
           shared_expert_gate_w):
    b, s, d = hidden_states.shape
    x = hidden_states.reshape(-1, d)
    t = x.shape[0]
    e = router_w.shape[0]
    f = expert_gate_w.shape[1]
    fs = shared_gate_w.shape[0]
    xT = x.T
    xTb = xT.astype(jnp.bfloat16)

    const = lambda i: (0, 0)
    outT, logitsT = pl.pallas_call(
        _moe_body,
        grid=(e,),
        in_specs=[
            pl.BlockSpec((d, t), const),
            pl.BlockSpec((d, t), const),
            pl.BlockSpec((e, d), const),
            pl.BlockSpec((1, f, d), lambda i: (i, 0, 0)),
            pl.BlockSpec((1, f, d), lambda i: (i, 0, 0)),
            pl.BlockSpec((1, d, f), lambda i: (i, 0, 0)),
            pl.BlockSpec((fs, d), const),
            pl.BlockSpec((fs, d), const),
            pl.BlockSpec((d, fs), const),
            pl.BlockSpec((1, d), const),
        ],
        out_specs=[
            pl.BlockSpec((d, t), const),
            pl.BlockSpec((e, t), const),
        ],
        out_shape=[
            jax.ShapeDtypeStruct((d, t), jnp.float32),
            jax.ShapeDtypeStruct((e, t), jnp.float32),
        ],
        scratch_shapes=[
            pltpu.VMEM((d, t), jnp.float32),
            pltpu.VMEM((e, t), jnp.float32),
        ],
        compiler_params=pltpu.CompilerParams(
            dimension_semantics=("arbitrary",),
        ),
    )(xT, xTb, router_w, expert_gate_w, expert_up_w, expert_down_w,
      shared_gate_w, shared_up_w, shared_down_w, shared_expert_gate_w)

    return outT.T.reshape(b, s, d), logitsT.T
